# baseline (speedup 1.0000x reference)
import functools

import jax
import jax.numpy as jnp
from jax import lax
from jax.experimental import pallas as pl
from jax.experimental.pallas import tpu as pltpu
from jax.experimental.pallas import tpu_sc as plsc

B = 4096
F = 26
ROWS = 100001
D = 64

NC = 2
NS = 16
NW = NC * NS

P = F // 2
TILE_B = 128
TILES_PER_P = B // TILE_B
NTILES = P * TILES_PER_P
TPW = NTILES // NW

_mesh = plsc.VectorSubcoreMesh(core_axis_name="c", subcore_axis_name="s")


@functools.partial(
    pl.kernel,
    mesh=_mesh,
    out_type=jax.ShapeDtypeStruct((B, F * D), jnp.float32),
    scratch_types=[
        pltpu.VMEM((2, TILE_B), jnp.int32),
        pltpu.VMEM((2, TILE_B, 2 * D), jnp.float32),
        pltpu.VMEM((TILE_B, 2 * D), jnp.float32),
        pltpu.VMEM((16, 2 * D), jnp.float32),
        pltpu.SemaphoreType.DMA((2,)),
    ],
    compiler_params=pltpu.CompilerParams(use_tc_tiling_on_sc=True),
)
def _gather_bias(tab_hbm, idx_hbm, bias_hbm, out_hbm,
                 idx_v, stage_v, tile_v, bias_v, sem):
    wid = lax.axis_index("s") * NC + lax.axis_index("c")
    pltpu.sync_copy(bias_hbm, bias_v)

    def tile_body(j, _):
        t = wid * TPW + j
        p = t // TILES_PER_P
        b0 = pl.multiple_of((t % TILES_PER_P) * TILE_B, TILE_B)

        def fire(half):
            f = 2 * p + half
            pltpu.sync_copy(
                idx_hbm.at[pl.ds(pl.multiple_of(f * B + b0, TILE_B), TILE_B)],
                idx_v.at[half])
            pltpu.async_copy(tab_hbm.at[idx_v.at[half]],
                             stage_v.at[half], sem.at[half])

        def process(half):
            pltpu.make_async_copy(
                tab_hbm.at[pl.ds(0, TILE_B)], stage_v.at[half],
                sem.at[half]).wait()
            b_regs = [bias_v[p, pl.ds(half * D + k * 16, 16)]
                      for k in range(4)]

            def row_body(r, _):
                for k in range(4):
                    tile_v[r, pl.ds(half * D + k * 16, 16)] = (
                        stage_v[half, r, pl.ds(half * D + k * 16, 16)]
                        + b_regs[k])
                return 0

            lax.fori_loop(0, TILE_B, row_body, 0)

        fire(0)
        fire(1)
        process(0)
        process(1)
        pltpu.sync_copy(
            tile_v,
            out_hbm.at[pl.ds(b0, TILE_B),
                       pl.ds(pl.multiple_of(p * 2 * D, 2 * D), 2 * D)])
        return 0

    lax.fori_loop(0, TPW, tile_body, 0)


def kernel(cat_features, tables, bias):
    offs = (jnp.arange(F, dtype=jnp.int32) // 2) * ROWS
    idx = (cat_features.T.astype(jnp.int32) + offs[:, None]).reshape(-1)
    tab_p = tables.reshape(P, 2, ROWS, D).transpose(0, 2, 1, 3).reshape(
        P * ROWS, 2 * D)
    bias_pairs = jnp.pad(bias.reshape(P, 2 * D), ((0, 3), (0, 0)))
    return _gather_bias(tab_p, idx, bias_pairs)


# speedup vs baseline: 35.9411x; 35.9411x over previous
import functools

import jax
import jax.numpy as jnp
from jax import lax
from jax.experimental import pallas as pl
from jax.experimental.pallas import tpu as pltpu
from jax.experimental.pallas import tpu_sc as plsc

B = 4096
F = 26
ROWS = 100001
D = 64

NC = 2
NS = 16
NW = NC * NS

P = F // 2
TILE_B = 128
TILES_PER_P = B // TILE_B
NTILES = P * TILES_PER_P
TPW = NTILES // NW

_mesh = plsc.VectorSubcoreMesh(core_axis_name="c", subcore_axis_name="s")


@functools.partial(
    pl.kernel,
    mesh=_mesh,
    out_type=jax.ShapeDtypeStruct((B, F * D), jnp.float32),
    scratch_types=[
        pltpu.VMEM((2, TILE_B), jnp.int32),
        pltpu.VMEM((2, TILE_B, 2 * D), jnp.float32),
        pltpu.VMEM((TILE_B, 2 * D), jnp.float32),
        pltpu.VMEM((16, 2 * D), jnp.float32),
        pltpu.SemaphoreType.DMA((2,)),
    ],
    compiler_params=pltpu.CompilerParams(use_tc_tiling_on_sc=True),
)
def _gather_bias(tab_hbm, idx_hbm, bias_hbm, out_hbm,
                 idx_v, stage_v, tile_v, bias_v, sem):
    wid = lax.axis_index("s") * NC + lax.axis_index("c")
    pltpu.sync_copy(bias_hbm, bias_v)

    def tile_body(j, _):
        t = wid * TPW + j
        p = t // TILES_PER_P
        b0 = pl.multiple_of((t % TILES_PER_P) * TILE_B, TILE_B)

        def fire(half):
            f = 2 * p + half
            pltpu.sync_copy(
                idx_hbm.at[pl.ds(pl.multiple_of(f * B + b0, TILE_B), TILE_B)],
                idx_v.at[half])
            pltpu.async_copy(tab_hbm.at[p].at[idx_v.at[half]],
                             stage_v.at[half], sem.at[half])

        def process(half):
            pltpu.make_async_copy(
                tab_hbm.at[p, pl.ds(0, TILE_B)], stage_v.at[half],
                sem.at[half]).wait()
            b_regs = [bias_v[p, pl.ds(half * D + k * 16, 16)]
                      for k in range(4)]

            def row_body(r, _):
                for k in range(4):
                    tile_v[r, pl.ds(half * D + k * 16, 16)] = (
                        stage_v[half, r, pl.ds(half * D + k * 16, 16)]
                        + b_regs[k])
                return 0

            lax.fori_loop(0, TILE_B, row_body, 0)

        fire(0)
        fire(1)
        process(0)
        process(1)
        pltpu.sync_copy(
            tile_v,
            out_hbm.at[pl.ds(b0, TILE_B),
                       pl.ds(pl.multiple_of(p * 2 * D, 2 * D), 2 * D)])
        return 0

    lax.fori_loop(0, TPW, tile_body, 0)


def kernel(cat_features, tables, bias):
    idx = cat_features.T.astype(jnp.int32).reshape(-1)
    tab_p = tables.reshape(P, 2, ROWS, D).transpose(0, 2, 1, 3).reshape(
        P, ROWS, 2 * D)
    bias_pairs = jnp.pad(bias.reshape(P, 2 * D), ((0, 3), (0, 0)))
    return _gather_bias(tab_p, idx, bias_pairs)


# speedup vs baseline: 36.3409x; 1.0111x over previous
import functools

import jax
import jax.numpy as jnp
from jax import lax
from jax.experimental import pallas as pl
from jax.experimental.pallas import tpu as pltpu
from jax.experimental.pallas import tpu_sc as plsc

B = 4096
F = 26
ROWS = 100001
D = 64

NC = 2
NS = 16
NW = NC * NS

P = F // 2
TILE_B = 128
TILES_PER_P = B // TILE_B
NTILES = P * TILES_PER_P
TPW = NTILES // NW

_mesh = plsc.VectorSubcoreMesh(core_axis_name="c", subcore_axis_name="s")


@functools.partial(
    pl.kernel,
    mesh=_mesh,
    out_type=jax.ShapeDtypeStruct((B, F * D), jnp.float32),
    scratch_types=[
        pltpu.VMEM((2, TILE_B), jnp.int32),
        pltpu.VMEM((2, TILE_B, 2 * D), jnp.float32),
        pltpu.VMEM((TILE_B, 2 * D), jnp.float32),
        pltpu.VMEM((16, 2 * D), jnp.float32),
        pltpu.SemaphoreType.DMA((2,)),
    ],
    compiler_params=pltpu.CompilerParams(use_tc_tiling_on_sc=True),
)
def _gather_bias(tab_hbm, idx_hbm, bias_hbm, out_hbm,
                 idx_v, stage_v, tile_v, bias_v, sem):
    wid = lax.axis_index("s") * NC + lax.axis_index("c")
    pltpu.sync_copy(bias_hbm, bias_v)

    def coords(j):
        t = wid * TPW + j
        p = t // TILES_PER_P
        b0 = pl.multiple_of((t % TILES_PER_P) * TILE_B, TILE_B)
        return p, b0

    def fire(p, b0, half):
        f = 2 * p + half
        pltpu.sync_copy(
            idx_hbm.at[pl.ds(pl.multiple_of(f * B + b0, TILE_B), TILE_B)],
            idx_v.at[half])
        pltpu.async_copy(tab_hbm.at[p].at[idx_v.at[half]],
                         stage_v.at[half], sem.at[half])

    def process(p, half):
        pltpu.make_async_copy(
            tab_hbm.at[p, pl.ds(0, TILE_B)], stage_v.at[half],
            sem.at[half]).wait()
        b_regs = [bias_v[p, pl.ds(half * D + k * 16, 16)] for k in range(4)]

        def row_body(r, _):
            for k in range(4):
                tile_v[r, pl.ds(half * D + k * 16, 16)] = (
                    stage_v[half, r, pl.ds(half * D + k * 16, 16)]
                    + b_regs[k])
            return 0

        lax.fori_loop(0, TILE_B, row_body, 0)

    p0, b00 = coords(0)
    fire(p0, b00, 0)

    def tile_body(j, _):
        p, b0 = coords(j)
        fire(p, b0, 1)
        process(p, 0)
        pn, b0n = coords(jnp.minimum(j + 1, TPW - 1))
        fire(pn, b0n, 0)
        process(p, 1)
        pltpu.sync_copy(
            tile_v,
            out_hbm.at[pl.ds(b0, TILE_B),
                       pl.ds(pl.multiple_of(p * 2 * D, 2 * D), 2 * D)])
        return 0

    lax.fori_loop(0, TPW, tile_body, 0)
    pl_, _b = coords(TPW - 1)
    pltpu.make_async_copy(
        tab_hbm.at[pl_, pl.ds(0, TILE_B)], stage_v.at[0], sem.at[0]).wait()


def kernel(cat_features, tables, bias):
    idx = cat_features.T.astype(jnp.int32).reshape(-1)
    tab_p = tables.reshape(P, 2, ROWS, D).transpose(0, 2, 1, 3).reshape(
        P, ROWS, 2 * D)
    bias_pairs = jnp.pad(bias.reshape(P, 2 * D), ((0, 3), (0, 0)))
    return _gather_bias(tab_p, idx, bias_pairs)
